# baseline (speedup 1.0000x reference)
import functools

import jax
import jax.numpy as jnp
from jax import lax
from jax.experimental import pallas as pl
from jax.experimental.pallas import tpu as pltpu
from jax.experimental.pallas import tpu_sc as plsc

NC = 2
NS = 16
NW = NC * NS
LANES = 16

L = 200
LP = 208
CHUNKS = LP // LANES
B = 1024
PB = B // NW
TW = PB * LP


def _make_kernel(vp):
    mesh = plsc.VectorSubcoreMesh(core_axis_name="c", subcore_axis_name="s")

    @functools.partial(
        pl.kernel,
        out_type=jax.ShapeDtypeStruct((B,), jnp.float32),
        mesh=mesh,
        scratch_types=[
            pltpu.VMEM((TW,), jnp.int32),
            pltpu.VMEM((vp,), jnp.float32),
            pltpu.VMEM((PB,), jnp.float32),
            pltpu.VMEM((LANES,), jnp.float32),
        ],
        compiler_params=pltpu.CompilerParams(needs_layout_passes=False),
    )
    def kern(text_hbm, w_hbm, b_hbm, out_hbm, tok_v, buf_v, out_v, bias_v):
        wid = lax.axis_index("s") * NC + lax.axis_index("c")
        base = wid * TW

        pltpu.sync_copy(text_hbm.at[pl.ds(base, TW)], tok_v)
        pltpu.sync_copy(b_hbm, bias_v)

        lane = lax.iota(jnp.int32, 16)
        lane_f = lane.astype(jnp.float32)
        pad_id = jnp.full((LANES,), vp - LANES, dtype=jnp.int32)

        def dedup_body(p, carry):
            off = p * LP
            for c in range(CHUNKS):
                idx = tok_v[pl.ds(off + c * LANES, LANES)]
                plsc.store_scatter(buf_v, [idx], lane_f + float(c * LANES))
            for c in range(CHUNKS):
                idx = tok_v[pl.ds(off + c * LANES, LANES)]
                tags = plsc.load_gather(buf_v, [idx])
                win = tags == lane_f + float(c * LANES)
                tok_v[pl.ds(off + c * LANES, LANES)] = jnp.where(
                    win, idx, pad_id)
            return carry

        lax.fori_loop(0, PB, dedup_body, jnp.int32(0))

        pltpu.sync_copy(w_hbm, buf_v)

        bias = bias_v[...]
        for g in range(PB // LANES):
            def sum_body(i, ovec):
                off = (g * LANES + i) * LP
                acc = jnp.zeros((LANES,), jnp.float32)
                for c in range(CHUNKS):
                    idx = tok_v[pl.ds(off + c * LANES, LANES)]
                    acc = acc + plsc.load_gather(buf_v, [idx])
                tot = jnp.sum(acc)
                return jnp.where(lane == i, tot, ovec)

            ovec = lax.fori_loop(0, LANES, sum_body,
                                 jnp.zeros((LANES,), jnp.float32))
            out_v[pl.ds(g * LANES, LANES)] = ovec + bias

        pltpu.sync_copy(out_v, out_hbm.at[pl.ds(wid * PB, PB)])

    return kern


def kernel(text, W, b):
    v = W.shape[1]
    vp = v + 2 * LANES
    pad = jnp.full((LP - L, B), v, dtype=jnp.int32)
    text_t = jnp.concatenate([text, pad], axis=0).T.reshape(-1)
    w_flat = jnp.concatenate([W[0], jnp.zeros((2 * LANES,), jnp.float32)])
    b16 = jnp.broadcast_to(b, (LANES,)).astype(jnp.float32)
    out = _make_kernel(vp)(text_t, w_flat, b16)
    return out.reshape(B, 1)
